# baseline (speedup 1.0000x reference)
import functools

import jax
import jax.numpy as jnp
from jax import lax
from jax.experimental import pallas as pl
from jax.experimental.pallas import tpu as pltpu
from jax.experimental.pallas import tpu_sc as plsc

NC = 2
NS = 16
NW = NC * NS
K = 128


def _sc_mesh():
    return plsc.VectorSubcoreMesh(core_axis_name="c", subcore_axis_name="s")


def _make_sc_deg(C, n_pad):
    ept = C * K
    nvec = ept // 16

    @functools.partial(
        pl.kernel,
        out_type=jax.ShapeDtypeStruct((NW, n_pad), jnp.float32),
        mesh=_sc_mesh(),
        compiler_params=pltpu.CompilerParams(needs_layout_passes=False),
        scratch_types=[
            pltpu.MemorySpace.VMEM((ept,), jnp.int32),
            pltpu.MemorySpace.VMEM((n_pad,), jnp.float32),
            pltpu.SemaphoreType.DMA,
        ],
    )
    def deg_kernel(dst_hbm, zeros_hbm, out_hbm, idx_v, hist, sem):
        c = lax.axis_index("c")
        s = lax.axis_index("s")
        wid = s * NC + c
        pltpu.sync_copy(dst_hbm.at[wid], idx_v)
        pltpu.sync_copy(zeros_hbm, hist)
        ones16 = jnp.full((16,), 1.0, jnp.float32)

        def body(i, carry):
            idx = idx_v[pl.ds(i * 16, 16)]
            plsc.addupdate_scatter(hist, [idx], ones16)
            return carry

        lax.fori_loop(0, nvec, body, 0)
        pltpu.sync_copy(hist, out_hbm.at[wid])

    return deg_kernel


def _make_sc_edges(C, n_pad, rows_pt, d):
    BLK = 8

    @functools.partial(
        pl.kernel,
        out_type=jax.ShapeDtypeStruct((NC, n_pad, d), jnp.float32),
        mesh=_sc_mesh(),
        scratch_types=[
            pltpu.MemorySpace.VMEM((C, K), jnp.int32),
            pltpu.MemorySpace.VMEM((BLK, K), jnp.int32),
            pltpu.MemorySpace.VMEM((K, d), jnp.float32),
            pltpu.MemorySpace.VMEM((K, d), jnp.float32),
            pltpu.MemorySpace.VMEM_SHARED((n_pad, d), jnp.float32),
            pltpu.SemaphoreType.DMA,
            pltpu.SemaphoreType.DMA,
        ],
    )
    def edge_kernel(hp_hbm, src_hbm, dst_hbm, zeros_hbm, out_hbm,
                    src_v, dst_blk, buf0, buf1, acc, sem0, sem1):
        c = lax.axis_index("c")
        s = lax.axis_index("s")
        wid = s * NC + c
        pltpu.sync_copy(src_hbm.at[wid], src_v)
        pltpu.sync_copy(zeros_hbm, acc.at[pl.ds(s * rows_pt, rows_pt)])
        plsc.subcore_barrier()

        pltpu.async_copy(hp_hbm.at[src_v.at[0]], buf0, sem0)

        def outer(b, carry):
            base = b * BLK
            pltpu.sync_copy(dst_hbm.at[wid, pl.ds(base, BLK)], dst_blk)
            for t in range(BLK // 2):
                j0 = base + 2 * t
                j1 = j0 + 1
                pltpu.async_copy(hp_hbm.at[src_v.at[j1]], buf1, sem1)
                pltpu.make_async_copy(hp_hbm.at[src_v.at[j0]], buf0, sem0).wait()
                pltpu.sync_copy(buf0, acc.at[dst_blk.at[2 * t]], add=True)
                jn = lax.rem(j0 + 2, C)
                pltpu.async_copy(hp_hbm.at[src_v.at[jn]], buf0, sem0)
                pltpu.make_async_copy(hp_hbm.at[src_v.at[j1]], buf1, sem1).wait()
                pltpu.sync_copy(buf1, acc.at[dst_blk.at[2 * t + 1]], add=True)
            return carry

        lax.fori_loop(0, C // BLK, outer, 0)
        pltpu.make_async_copy(hp_hbm.at[src_v.at[0]], buf0, sem0).wait()
        plsc.subcore_barrier()
        pltpu.sync_copy(
            acc.at[pl.ds(s * rows_pt, rows_pt)],
            out_hbm.at[c, pl.ds(s * rows_pt, rows_pt)],
        )

    return edge_kernel


def _tc_call(fn, out_shape):
    return pl.pallas_call(fn, out_shape=out_shape)


def _deg_col(cnt, n):
    ones_w = jnp.ones((NW, 1), jnp.float32)
    col = lax.dot_general(cnt, ones_w, (((0,), (0,)), ((), ())),
                          preferred_element_type=jnp.float32)
    return 1.0 + col[0:n, :]


def _make_tc1(n, din, d):
    def body(x_ref, w_ref, cnt_ref, hp_ref):
        dis = lax.rsqrt(_deg_col(cnt_ref[...], n))
        h = jnp.dot(x_ref[...], w_ref[...], preferred_element_type=jnp.float32)
        hp_ref[...] = h * dis

    return _tc_call(body, jax.ShapeDtypeStruct((n, d), jnp.float32))


def _make_tc2(n, d, dout):
    def body(acc_ref, hp_ref, cnt_ref, w_ref, b_ref, hp2_ref):
        dis = lax.rsqrt(_deg_col(cnt_ref[...], n))
        t = (acc_ref[0, 0:n, :] + acc_ref[1, 0:n, :] + hp_ref[...]) * dis + b_ref[...]
        y = jnp.maximum(t, 0.0)
        h2 = jnp.dot(y, w_ref[...], preferred_element_type=jnp.float32)
        hp2_ref[...] = h2 * dis

    return _tc_call(body, jax.ShapeDtypeStruct((n, dout), jnp.float32))


def _make_tc3(n, d):
    def body(acc_ref, hp_ref, cnt_ref, b_ref, out_ref):
        dis = lax.rsqrt(_deg_col(cnt_ref[...], n))
        out_ref[...] = (
            acc_ref[0, 0:n, :] + acc_ref[1, 0:n, :] + hp_ref[...]
        ) * dis + b_ref[...]

    return _tc_call(body, jax.ShapeDtypeStruct((n, d), jnp.float32))


def kernel(x, edge_index, W1, b1, W2, b2):
    n, din = x.shape
    dh = W1.shape[1]
    dout = W2.shape[1]
    e = edge_index.shape[1]

    C = -(-e // (NW * K))
    C = -(-C // 8) * 8
    e_pad = NW * C * K
    rows_pt = -(-(n + 1) // (NS * 8)) * 8
    n_pad = rows_pt * NS

    src = edge_index[0]
    dst = edge_index[1]
    pad = e_pad - e
    dump = n + jnp.arange(pad, dtype=jnp.int32) % (n_pad - n)
    src_r = jnp.concatenate([src, jnp.zeros((pad,), jnp.int32)]).reshape(NW, C, K)
    dst_r = jnp.concatenate([dst, dump]).reshape(NW, C, K)
    dst_flat = dst_r.reshape(NW, C * K)

    zerosn = jnp.zeros((n_pad,), jnp.float32)
    zerosd = jnp.zeros((rows_pt, dh), jnp.float32)
    b1r = b1.reshape(1, dh)
    b2r = b2.reshape(1, dout)

    sc_deg = _make_sc_deg(C, n_pad)
    sc_edges = _make_sc_edges(C, n_pad, rows_pt, dh)
    tc1 = _make_tc1(n, din, dh)
    tc2 = _make_tc2(n, dh, dout)
    tc3 = _make_tc3(n, dh)

    cnt = sc_deg(dst_flat, zerosn)
    hp1 = tc1(x, W1, cnt)
    acc1 = sc_edges(hp1, src_r, dst_r, zerosd)
    hp2 = tc2(acc1, hp1, cnt, W2, b1r)
    acc2 = sc_edges(hp2, src_r, dst_r, zerosd)
    out = tc3(acc2, hp2, cnt, b2r)
    return out


# speedup vs baseline: 3.6811x; 3.6811x over previous
import functools

import jax
import jax.numpy as jnp
from jax import lax
from jax.experimental import pallas as pl
from jax.experimental.pallas import tpu as pltpu
from jax.experimental.pallas import tpu_sc as plsc

NC = 2
NS = 16
NW = NC * NS
K = 128


def _sc_mesh():
    return plsc.VectorSubcoreMesh(core_axis_name="c", subcore_axis_name="s")


def _make_sc_deg(C, n_pad):
    ept = C * K
    nvec = ept // 16

    @functools.partial(
        pl.kernel,
        out_type=jax.ShapeDtypeStruct((NW, n_pad), jnp.float32),
        mesh=_sc_mesh(),
        compiler_params=pltpu.CompilerParams(needs_layout_passes=False),
        scratch_types=[
            pltpu.MemorySpace.VMEM((ept,), jnp.int32),
            pltpu.MemorySpace.VMEM((n_pad,), jnp.float32),
            pltpu.SemaphoreType.DMA,
        ],
    )
    def deg_kernel(dst_hbm, zeros_hbm, out_hbm, idx_v, hist, sem):
        c = lax.axis_index("c")
        s = lax.axis_index("s")
        wid = s * NC + c
        pltpu.sync_copy(dst_hbm.at[wid], idx_v)
        pltpu.sync_copy(zeros_hbm, hist)
        ones16 = jnp.full((16,), 1.0, jnp.float32)

        def body(i, carry):
            idx = idx_v[pl.ds(i * 16, 16)]
            plsc.addupdate_scatter(hist, [idx], ones16)
            return carry

        lax.fori_loop(0, nvec, body, 0)
        pltpu.sync_copy(hist, out_hbm.at[wid])

    return deg_kernel


def _make_sc_edges(C, n_pad, rows_pt, d):
    BLK = 8

    @functools.partial(
        pl.kernel,
        out_type=jax.ShapeDtypeStruct((NC, n_pad, d), jnp.float32),
        mesh=_sc_mesh(),
        scratch_types=[
            pltpu.MemorySpace.VMEM((C, K), jnp.int32),
            pltpu.MemorySpace.VMEM((BLK, K), jnp.int32),
            pltpu.MemorySpace.VMEM((K, d), jnp.float32),
            pltpu.MemorySpace.VMEM((K, d), jnp.float32),
            pltpu.MemorySpace.VMEM_SHARED((n_pad, d), jnp.float32),
            pltpu.SemaphoreType.DMA,
            pltpu.SemaphoreType.DMA,
        ],
    )
    def edge_kernel(hp_hbm, src_hbm, dst_hbm, zeros_hbm, out_hbm,
                    src_v, dst_blk, buf0, buf1, acc, sem0, sem1):
        c = lax.axis_index("c")
        s = lax.axis_index("s")
        wid = s * NC + c
        pltpu.sync_copy(src_hbm.at[wid], src_v)
        pltpu.sync_copy(zeros_hbm, acc.at[pl.ds(s * rows_pt, rows_pt)])
        plsc.subcore_barrier()

        pltpu.async_copy(hp_hbm.at[src_v.at[0]], buf0, sem0)

        def outer(b, carry):
            base = b * BLK
            pltpu.sync_copy(dst_hbm.at[wid, pl.ds(base, BLK)], dst_blk)
            for t in range(BLK // 2):
                j0 = base + 2 * t
                j1 = j0 + 1
                pltpu.async_copy(hp_hbm.at[src_v.at[j1]], buf1, sem1)
                pltpu.make_async_copy(hp_hbm.at[src_v.at[j0]], buf0, sem0).wait()
                pltpu.sync_copy(buf0, acc.at[dst_blk.at[2 * t]], add=True)
                jn = lax.rem(j0 + 2, C)
                pltpu.async_copy(hp_hbm.at[src_v.at[jn]], buf0, sem0)
                pltpu.make_async_copy(hp_hbm.at[src_v.at[j1]], buf1, sem1).wait()
                pltpu.sync_copy(buf1, acc.at[dst_blk.at[2 * t + 1]], add=True)
            return carry

        lax.fori_loop(0, C // BLK, outer, 0)
        pltpu.make_async_copy(hp_hbm.at[src_v.at[0]], buf0, sem0).wait()
        plsc.subcore_barrier()
        pltpu.sync_copy(
            acc.at[pl.ds(s * rows_pt, rows_pt)],
            out_hbm.at[c, pl.ds(s * rows_pt, rows_pt)],
        )

    return edge_kernel


def _tc_call(fn, out_shape):
    return pl.pallas_call(fn, out_shape=out_shape)


def _deg_col(cnt, n):
    ones_w = jnp.ones((NW, 1), jnp.float32)
    col = lax.dot_general(cnt, ones_w, (((0,), (0,)), ((), ())),
                          preferred_element_type=jnp.float32)
    return 1.0 + col[0:n, :]


def _make_tc1(n, din, d):
    def body(x_ref, w_ref, cnt_ref, hp_ref):
        dis = lax.rsqrt(_deg_col(cnt_ref[...], n))
        h = jnp.dot(x_ref[...], w_ref[...], preferred_element_type=jnp.float32)
        hp_ref[...] = h * dis

    return _tc_call(body, jax.ShapeDtypeStruct((n, d), jnp.float32))


def _make_tc2(n, d, dout):
    def body(acc_ref, hp_ref, cnt_ref, w_ref, b_ref, hp2_ref):
        dis = lax.rsqrt(_deg_col(cnt_ref[...], n))
        t = (acc_ref[0, 0:n, :] + acc_ref[1, 0:n, :] + hp_ref[...]) * dis + b_ref[...]
        y = jnp.maximum(t, 0.0)
        h2 = jnp.dot(y, w_ref[...], preferred_element_type=jnp.float32)
        hp2_ref[...] = h2 * dis

    return _tc_call(body, jax.ShapeDtypeStruct((n, dout), jnp.float32))


def _make_tc3(n, d):
    def body(acc_ref, hp_ref, cnt_ref, b_ref, out_ref):
        dis = lax.rsqrt(_deg_col(cnt_ref[...], n))
        out_ref[...] = (
            acc_ref[0, 0:n, :] + acc_ref[1, 0:n, :] + hp_ref[...]
        ) * dis + b_ref[...]

    return _tc_call(body, jax.ShapeDtypeStruct((n, d), jnp.float32))


def kernel(x, edge_index, W1, b1, W2, b2):
    n, din = x.shape
    dh = W1.shape[1]
    dout = W2.shape[1]
    e = edge_index.shape[1]

    C = -(-e // (NW * K))
    C = -(-C // 8) * 8
    e_pad = NW * C * K
    rows_pt = -(-(n + 1) // (NS * 8)) * 8
    n_pad = rows_pt * NS

    src = edge_index[0]
    dst = edge_index[1]
    pad = e_pad - e
    dump = n + jnp.arange(pad, dtype=jnp.int32) % (n_pad - n)
    srcpad = jnp.arange(pad, dtype=jnp.int32) % n
    src_r = jnp.concatenate([src, srcpad]).reshape(NW, C, K)
    dst_r = jnp.concatenate([dst, dump]).reshape(NW, C, K)
    dst_flat = dst_r.reshape(NW, C * K)

    zerosn = jnp.zeros((n_pad,), jnp.float32)
    zerosd = jnp.zeros((rows_pt, dh), jnp.float32)
    b1r = b1.reshape(1, dh)
    b2r = b2.reshape(1, dout)

    sc_deg = _make_sc_deg(C, n_pad)
    sc_edges = _make_sc_edges(C, n_pad, rows_pt, dh)
    tc1 = _make_tc1(n, din, dh)
    tc2 = _make_tc2(n, dh, dout)
    tc3 = _make_tc3(n, dh)

    cnt = sc_deg(dst_flat, zerosn)
    hp1 = tc1(x, W1, cnt)
    acc1 = sc_edges(hp1, src_r, dst_r, zerosd)
    hp2 = tc2(acc1, hp1, cnt, W2, b1r)
    acc2 = sc_edges(hp2, src_r, dst_r, zerosd)
    out = tc3(acc2, hp2, cnt, b2r)
    return out


# speedup vs baseline: 3.7711x; 1.0244x over previous
import functools

import jax
import jax.numpy as jnp
from jax import lax
from jax.experimental import pallas as pl
from jax.experimental.pallas import tpu as pltpu
from jax.experimental.pallas import tpu_sc as plsc

NC = 2
NS = 16
NW = NC * NS
K = 64


def _sc_mesh():
    return plsc.VectorSubcoreMesh(core_axis_name="c", subcore_axis_name="s")


def _make_sc_deg(C, n_pad):
    ept = C * K
    nvec = ept // 16

    @functools.partial(
        pl.kernel,
        out_type=jax.ShapeDtypeStruct((NW, n_pad), jnp.float32),
        mesh=_sc_mesh(),
        compiler_params=pltpu.CompilerParams(needs_layout_passes=False),
        scratch_types=[
            pltpu.MemorySpace.VMEM((ept,), jnp.int32),
            pltpu.MemorySpace.VMEM((n_pad,), jnp.float32),
            pltpu.SemaphoreType.DMA,
        ],
    )
    def deg_kernel(dst_hbm, zeros_hbm, out_hbm, idx_v, hist, sem):
        c = lax.axis_index("c")
        s = lax.axis_index("s")
        wid = s * NC + c
        pltpu.sync_copy(dst_hbm.at[wid], idx_v)
        pltpu.sync_copy(zeros_hbm, hist)
        ones16 = jnp.full((16,), 1.0, jnp.float32)

        def body(i, carry):
            idx = idx_v[pl.ds(i * 16, 16)]
            plsc.addupdate_scatter(hist, [idx], ones16)
            return carry

        lax.fori_loop(0, nvec, body, 0)
        pltpu.sync_copy(hist, out_hbm.at[wid])

    return deg_kernel


def _make_sc_edges(C, n_pad, rows_pt, d):
    NB = 4

    CH = C // 2

    @functools.partial(
        pl.kernel,
        out_type=jax.ShapeDtypeStruct((NC, n_pad, d), jnp.float32),
        mesh=_sc_mesh(),
        scratch_types=[
            pltpu.MemorySpace.VMEM((CH, 2 * K), jnp.int32),
            pltpu.MemorySpace.VMEM((NB, K), jnp.int32),
            [pltpu.MemorySpace.VMEM((K, d), jnp.float32) for _ in range(NB)],
            pltpu.MemorySpace.VMEM_SHARED((n_pad, d), jnp.float32),
            [pltpu.SemaphoreType.DMA for _ in range(NB)],
        ],
    )
    def edge_kernel(hp_hbm, src_hbm, dst_hbm, zeros_hbm, out_hbm,
                    src_v, dst_blk, bufs, acc, sems):
        c = lax.axis_index("c")
        s = lax.axis_index("s")
        wid = s * NC + c
        pltpu.sync_copy(src_hbm.at[wid], src_v)
        pltpu.sync_copy(zeros_hbm, acc.at[pl.ds(s * rows_pt, rows_pt)])
        plsc.subcore_barrier()

        def src_idx(row, half):
            return src_v.at[row, pl.ds(half * K, K)]

        for t in range(NB):
            pltpu.async_copy(hp_hbm.at[src_idx(t // 2, t % 2)], bufs[t], sems[t])

        def outer(b, carry):
            base = b * NB
            pltpu.sync_copy(dst_hbm.at[wid, pl.ds(base, NB)], dst_blk)
            for t in range(NB):
                pltpu.make_async_copy(hp_hbm.at[src_idx(0, 0)], bufs[t], sems[t]).wait()
                pltpu.sync_copy(bufs[t], acc.at[dst_blk.at[t]], add=True)
                rn = lax.rem(2 * b + (t + NB) // 2, CH)
                pltpu.async_copy(hp_hbm.at[src_idx(rn, t % 2)], bufs[t], sems[t])
            return carry

        lax.fori_loop(0, C // NB, outer, 0)
        for t in range(NB):
            pltpu.make_async_copy(hp_hbm.at[src_idx(0, 0)], bufs[t], sems[t]).wait()
        plsc.subcore_barrier()
        pltpu.sync_copy(
            acc.at[pl.ds(s * rows_pt, rows_pt)],
            out_hbm.at[c, pl.ds(s * rows_pt, rows_pt)],
        )

    return edge_kernel


def _tc_call(fn, out_shape):
    return pl.pallas_call(fn, out_shape=out_shape)


def _make_tc1(n, din, d):
    def body(x_ref, w_ref, cnt_ref, hp_ref, dis_ref):
        ones_w = jnp.ones((NW, 1), jnp.float32)
        col = lax.dot_general(cnt_ref[...], ones_w, (((0,), (0,)), ((), ())),
                              preferred_element_type=jnp.float32)
        dis = lax.rsqrt(1.0 + col[0:n, :])
        dis_ref[...] = dis
        h = jnp.dot(x_ref[...], w_ref[...], preferred_element_type=jnp.float32)
        hp_ref[...] = h * dis

    return _tc_call(body, (jax.ShapeDtypeStruct((n, d), jnp.float32),
                           jax.ShapeDtypeStruct((n, 1), jnp.float32)))


def _make_tc2(n, d, dout):
    def body(acc_ref, hp_ref, dis_ref, w_ref, b_ref, hp2_ref):
        dis = dis_ref[...]
        t = (acc_ref[0, 0:n, :] + acc_ref[1, 0:n, :] + hp_ref[...]) * dis + b_ref[...]
        y = jnp.maximum(t, 0.0)
        h2 = jnp.dot(y, w_ref[...], preferred_element_type=jnp.float32)
        hp2_ref[...] = h2 * dis

    return _tc_call(body, jax.ShapeDtypeStruct((n, dout), jnp.float32))


def _make_tc3(n, d):
    def body(acc_ref, hp_ref, dis_ref, b_ref, out_ref):
        out_ref[...] = (
            acc_ref[0, 0:n, :] + acc_ref[1, 0:n, :] + hp_ref[...]
        ) * dis_ref[...] + b_ref[...]

    return _tc_call(body, jax.ShapeDtypeStruct((n, d), jnp.float32))


def kernel(x, edge_index, W1, b1, W2, b2):
    n, din = x.shape
    dh = W1.shape[1]
    dout = W2.shape[1]
    e = edge_index.shape[1]

    C = -(-e // (NW * K))
    C = -(-C // 8) * 8
    e_pad = NW * C * K
    rows_pt = -(-(n + 1) // (NS * 8)) * 8
    n_pad = rows_pt * NS

    src = edge_index[0]
    dst = edge_index[1]
    pad = e_pad - e
    dump = n + jnp.arange(pad, dtype=jnp.int32) % (n_pad - n)
    srcpad = jnp.arange(pad, dtype=jnp.int32) % n
    src_r = jnp.concatenate([src, srcpad]).reshape(NW, C // 2, 2 * K)
    dst_r = jnp.concatenate([dst, dump]).reshape(NW, C, K)
    dst_flat = dst_r.reshape(NW, C * K)

    zerosn = jnp.zeros((n_pad,), jnp.float32)
    zerosd = jnp.zeros((rows_pt, dh), jnp.float32)
    b1r = b1.reshape(1, dh)
    b2r = b2.reshape(1, dout)

    sc_deg = _make_sc_deg(C, n_pad)
    sc_edges = _make_sc_edges(C, n_pad, rows_pt, dh)
    tc1 = _make_tc1(n, din, dh)
    tc2 = _make_tc2(n, dh, dout)
    tc3 = _make_tc3(n, dh)

    cnt = sc_deg(dst_flat, zerosn)
    hp1, dis = tc1(x, W1, cnt)
    acc1 = sc_edges(hp1, src_r, dst_r, zerosd)
    hp2 = tc2(acc1, hp1, dis, W2, b1r)
    acc2 = sc_edges(hp2, src_r, dst_r, zerosd)
    out = tc3(acc2, hp2, dis, b2r)
    return out
